# Initial kernel scaffold; baseline (speedup 1.0000x reference)
import functools

import jax
import jax.numpy as jnp
from jax import lax
from jax.experimental import pallas as pl
from jax.experimental.pallas import tpu as pltpu
from jax.experimental.pallas import tpu_sc as plsc

N_NODES = 10000
N_EDGES = 320000
D = 128

NC = 2
NS = 16
NW = NC * NS

CHUNK = 128
E_PAD = 327680
STEPS = E_PAD // (NW * CHUNK)
N_PAD = 10240
ROWS_PER_TILE = N_PAD // NS

W1 = 144
W2 = 128

ROW_BLOCK = 1000


def _dotT(a, w):
    return lax.dot_general(a, w, (((1,), (1,)), ((), ())),
                           preferred_element_type=jnp.float32)


def _tc1_body(x_ref, wext_ref, bext_ref, wr_ref, br_ref, yext_ref, z_ref):
    x = x_ref[...]
    yext_ref[...] = _dotT(x, wext_ref[...]) + bext_ref[...]
    z_ref[...] = _dotT(x, wr_ref[...]) + br_ref[...]


def _tc2_body(acc_ref, z1_ref, wl2_ref, bl2_ref, wr2_ref,
              y2_ref, z2_ref, rinv_ref):
    a = acc_ref[0] + acc_ref[1]
    cnt = a[:, 128][:, None]
    rinv = 1.0 / jnp.maximum(cnt, 1.0)
    h = jnp.maximum(a[:, :D] * rinv + z1_ref[...], 0.0)
    y2_ref[...] = _dotT(h, wl2_ref[...])
    z2_ref[...] = _dotT(h, wr2_ref[...]) + bl2_ref[...]
    rinv_ref[...] = jnp.broadcast_to(rinv, (ROW_BLOCK, D))


def _tc3_body(acc_ref, z2_ref, rinv_ref, out_ref):
    out_ref[...] = (acc_ref[0] + acc_ref[1]) * rinv_ref[...] + z2_ref[...]


def _tc1(x, w_ext, b_ext, w_r, b_r):
    grid = (N_NODES // ROW_BLOCK,)
    return pl.pallas_call(
        _tc1_body,
        grid=grid,
        in_specs=[
            pl.BlockSpec((ROW_BLOCK, D), lambda i: (i, 0)),
            pl.BlockSpec((W1, D), lambda i: (0, 0)),
            pl.BlockSpec((1, W1), lambda i: (0, 0)),
            pl.BlockSpec((D, D), lambda i: (0, 0)),
            pl.BlockSpec((1, D), lambda i: (0, 0)),
        ],
        out_specs=[
            pl.BlockSpec((ROW_BLOCK, W1), lambda i: (i, 0)),
            pl.BlockSpec((ROW_BLOCK, D), lambda i: (i, 0)),
        ],
        out_shape=[
            jax.ShapeDtypeStruct((N_NODES, W1), jnp.float32),
            jax.ShapeDtypeStruct((N_NODES, D), jnp.float32),
        ],
    )(x, w_ext, b_ext, w_r, b_r)


def _tc2(acc1, z1, w_l2, b_l2, w_r2):
    grid = (N_NODES // ROW_BLOCK,)
    return pl.pallas_call(
        _tc2_body,
        grid=grid,
        in_specs=[
            pl.BlockSpec((NC, ROW_BLOCK, W1), lambda i: (0, i, 0)),
            pl.BlockSpec((ROW_BLOCK, D), lambda i: (i, 0)),
            pl.BlockSpec((D, D), lambda i: (0, 0)),
            pl.BlockSpec((1, D), lambda i: (0, 0)),
            pl.BlockSpec((D, D), lambda i: (0, 0)),
        ],
        out_specs=[
            pl.BlockSpec((ROW_BLOCK, D), lambda i: (i, 0)),
            pl.BlockSpec((ROW_BLOCK, D), lambda i: (i, 0)),
            pl.BlockSpec((ROW_BLOCK, D), lambda i: (i, 0)),
        ],
        out_shape=[
            jax.ShapeDtypeStruct((N_NODES, D), jnp.float32),
            jax.ShapeDtypeStruct((N_NODES, D), jnp.float32),
            jax.ShapeDtypeStruct((N_NODES, D), jnp.float32),
        ],
    )(acc1, z1, w_l2, b_l2, w_r2)


def _tc3(acc2, z2, rinv):
    grid = (N_NODES // ROW_BLOCK,)
    return pl.pallas_call(
        _tc3_body,
        grid=grid,
        in_specs=[
            pl.BlockSpec((NC, ROW_BLOCK, W2), lambda i: (0, i, 0)),
            pl.BlockSpec((ROW_BLOCK, D), lambda i: (i, 0)),
            pl.BlockSpec((ROW_BLOCK, D), lambda i: (i, 0)),
        ],
        out_specs=pl.BlockSpec((ROW_BLOCK, D), lambda i: (i, 0)),
        out_shape=jax.ShapeDtypeStruct((N_NODES, D), jnp.float32),
    )(acc2, z2, rinv)


def _make_sc_scatter(width):
    mesh = plsc.VectorSubcoreMesh(core_axis_name="c", subcore_axis_name="s",
                                  num_cores=NC, num_subcores=NS)

    @functools.partial(
        pl.kernel,
        out_type=jax.ShapeDtypeStruct((NC, N_PAD, width), jnp.float32),
        mesh=mesh,
        scratch_types=[
            pltpu.VMEM((STEPS, CHUNK), jnp.int32),
            pltpu.VMEM((STEPS, CHUNK), jnp.int32),
            pltpu.VMEM((CHUNK, width), jnp.float32),
            pltpu.VMEM_SHARED((N_PAD, width), jnp.float32),
            pltpu.SemaphoreType.DMA,
        ],
    )
    def sc_kernel(y_hbm, src_hbm, dst_hbm, zeros_hbm, out_hbm,
                  src_v, dst_v, rows_v, acc, gsem):
        c = lax.axis_index("c")
        s = lax.axis_index("s")
        wid = s * NC + c

        pltpu.sync_copy(src_hbm.at[wid], src_v)
        pltpu.sync_copy(dst_hbm.at[wid], dst_v)
        pltpu.sync_copy(zeros_hbm, acc.at[pl.ds(s * ROWS_PER_TILE,
                                                ROWS_PER_TILE)])
        plsc.subcore_barrier()

        def step(j, _):
            pltpu.async_copy(y_hbm.at[src_v.at[j]], rows_v, gsem).wait()
            pltpu.sync_copy(rows_v, acc.at[dst_v.at[j]], add=True)
            return 0

        lax.fori_loop(0, STEPS, step, 0)
        plsc.subcore_barrier()

        pltpu.sync_copy(
            acc.at[pl.ds(s * ROWS_PER_TILE, ROWS_PER_TILE)],
            out_hbm.at[c].at[pl.ds(s * ROWS_PER_TILE, ROWS_PER_TILE)])

    return sc_kernel


_sc_scatter_w1 = _make_sc_scatter(W1)
_sc_scatter_w2 = _make_sc_scatter(W2)


@jax.jit
def kernel(x_dict, edge_index, W_l1, b_l1, W_r1, W_l2, b_l2, W_r2):
    x = x_dict
    n_fill = E_PAD - N_EDGES
    src = jnp.concatenate([edge_index[0],
                           jnp.zeros((n_fill,), jnp.int32)])
    dst = jnp.concatenate([edge_index[1],
                           jnp.full((n_fill,), N_NODES, jnp.int32)])
    src = src.reshape(NW, STEPS, CHUNK)
    dst = dst.reshape(NW, STEPS, CHUNK)

    w_ext = jnp.concatenate([W_l1, jnp.zeros((W1 - D, D), jnp.float32)], axis=0)
    b_ext = jnp.zeros((1, W1), jnp.float32).at[0, D].set(1.0)

    y1, z1 = _tc1(x, w_ext, b_ext, W_r1, b_l1.reshape(1, D))

    zeros1 = jnp.zeros((ROWS_PER_TILE, W1), jnp.float32)
    acc1 = _sc_scatter_w1(y1, src, dst, zeros1)

    y2, z2, rinv = _tc2(acc1, z1, W_l2, b_l2.reshape(1, D), W_r2)

    zeros2 = jnp.zeros((ROWS_PER_TILE, W2), jnp.float32)
    acc2 = _sc_scatter_w2(y2, src, dst, zeros2)

    return _tc3(acc2, z2, rinv)


# speedup vs baseline: 8.5940x; 8.5940x over previous
import functools

import jax
import jax.numpy as jnp
from jax import lax
from jax.experimental import pallas as pl
from jax.experimental.pallas import tpu as pltpu
from jax.experimental.pallas import tpu_sc as plsc

N_NODES = 10000
N_EDGES = 320000
D = 128
L = 16

NC = 2
NS = 16
NW = NC * NS

CHUNK = 128
E_PAD = 327680
STEPS = E_PAD // (NW * CHUNK)
N_PAD = 10240
ROWS_PER_TILE = N_PAD // NS
CROWS = N_PAD // CHUNK
CROWS_PER_TILE = CROWS // NS

ROW_BLOCK = 1000


def _dotT(a, w):
    return lax.dot_general(a, w, (((1,), (1,)), ((), ())),
                           preferred_element_type=jnp.float32)


def _tc1_body(x_ref, wl_ref, wr_ref, br_ref, y_ref, z_ref):
    x = x_ref[...]
    y_ref[...] = _dotT(x, wl_ref[...])
    z_ref[...] = _dotT(x, wr_ref[...]) + br_ref[...]


def _tc2_body(acc_ref, cnt_ref, z1_ref, wl2_ref, bl2_ref, wr2_ref,
              y2_ref, z2_ref, rinv_ref):
    a = acc_ref[0] + acc_ref[1]
    cnt = (cnt_ref[0] + cnt_ref[1])
    rinv = 1.0 / jnp.maximum(cnt, 1.0)
    h = jnp.maximum(a * rinv + z1_ref[...], 0.0)
    y2_ref[...] = _dotT(h, wl2_ref[...])
    z2_ref[...] = _dotT(h, wr2_ref[...]) + bl2_ref[...]
    rinv_ref[...] = jnp.broadcast_to(rinv, (ROW_BLOCK, D))


def _tc3_body(acc_ref, z2_ref, rinv_ref, out_ref):
    out_ref[...] = (acc_ref[0] + acc_ref[1]) * rinv_ref[...] + z2_ref[...]


def _tc1(x, w_l, w_r, b_r):
    grid = (N_NODES // ROW_BLOCK,)
    return pl.pallas_call(
        _tc1_body,
        grid=grid,
        in_specs=[
            pl.BlockSpec((ROW_BLOCK, D), lambda i: (i, 0)),
            pl.BlockSpec((D, D), lambda i: (0, 0)),
            pl.BlockSpec((D, D), lambda i: (0, 0)),
            pl.BlockSpec((1, D), lambda i: (0, 0)),
        ],
        out_specs=[
            pl.BlockSpec((ROW_BLOCK, D), lambda i: (i, 0)),
            pl.BlockSpec((ROW_BLOCK, D), lambda i: (i, 0)),
        ],
        out_shape=[
            jax.ShapeDtypeStruct((N_NODES, D), jnp.float32),
            jax.ShapeDtypeStruct((N_NODES, D), jnp.float32),
        ],
    )(x, w_l, w_r, b_r)


def _tc2(acc1, cnt, z1, w_l2, b_l2, w_r2):
    grid = (N_NODES // ROW_BLOCK,)
    return pl.pallas_call(
        _tc2_body,
        grid=grid,
        in_specs=[
            pl.BlockSpec((NC, ROW_BLOCK, D), lambda i: (0, i, 0)),
            pl.BlockSpec((NC, ROW_BLOCK, 1), lambda i: (0, i, 0)),
            pl.BlockSpec((ROW_BLOCK, D), lambda i: (i, 0)),
            pl.BlockSpec((D, D), lambda i: (0, 0)),
            pl.BlockSpec((1, D), lambda i: (0, 0)),
            pl.BlockSpec((D, D), lambda i: (0, 0)),
        ],
        out_specs=[
            pl.BlockSpec((ROW_BLOCK, D), lambda i: (i, 0)),
            pl.BlockSpec((ROW_BLOCK, D), lambda i: (i, 0)),
            pl.BlockSpec((ROW_BLOCK, D), lambda i: (i, 0)),
        ],
        out_shape=[
            jax.ShapeDtypeStruct((N_NODES, D), jnp.float32),
            jax.ShapeDtypeStruct((N_NODES, D), jnp.float32),
            jax.ShapeDtypeStruct((N_NODES, D), jnp.float32),
        ],
    )(acc1, cnt, z1, w_l2, b_l2, w_r2)


def _tc3(acc2, z2, rinv):
    grid = (N_NODES // ROW_BLOCK,)
    return pl.pallas_call(
        _tc3_body,
        grid=grid,
        in_specs=[
            pl.BlockSpec((NC, ROW_BLOCK, D), lambda i: (0, i, 0)),
            pl.BlockSpec((ROW_BLOCK, D), lambda i: (i, 0)),
            pl.BlockSpec((ROW_BLOCK, D), lambda i: (i, 0)),
        ],
        out_specs=pl.BlockSpec((ROW_BLOCK, D), lambda i: (i, 0)),
        out_shape=jax.ShapeDtypeStruct((N_NODES, D), jnp.float32),
    )(acc2, z2, rinv)


def _make_sc_scatter(with_count):
    mesh = plsc.VectorSubcoreMesh(core_axis_name="c", subcore_axis_name="s",
                                  num_cores=NC, num_subcores=NS)

    out_type = [jax.ShapeDtypeStruct((NC, N_PAD, D), jnp.float32)]
    scratch = [
        pltpu.VMEM((STEPS, CHUNK), jnp.int32),
        pltpu.VMEM((STEPS, CHUNK), jnp.int32),
        pltpu.VMEM((CHUNK, D), jnp.float32),
        pltpu.VMEM_SHARED((N_PAD, D), jnp.float32),
        pltpu.SemaphoreType.DMA,
    ]
    if with_count:
        out_type.append(jax.ShapeDtypeStruct((NC, CROWS, CHUNK), jnp.float32))
        scratch += [
            pltpu.VMEM((CROWS, CHUNK), jnp.float32),
            pltpu.VMEM((CROWS,), jnp.int32),
            pltpu.VMEM_SHARED((CROWS, CHUNK), jnp.float32),
        ]

    @functools.partial(
        pl.kernel, out_type=out_type, mesh=mesh, scratch_types=scratch,
        compiler_params=pltpu.CompilerParams(needs_layout_passes=False))
    def sc_kernel(y_hbm, src_hbm, dst_hbm, zeros_hbm, *refs):
        if with_count:
            (out_hbm, cnt_hbm, src_v, dst_v, rows_v, acc, gsem,
             hist_v, iden_v, hist_sh) = refs
        else:
            out_hbm, src_v, dst_v, rows_v, acc, gsem = refs

        c = lax.axis_index("c")
        s = lax.axis_index("s")
        wid = s * NC + c

        pltpu.sync_copy(src_hbm.at[wid], src_v)
        pltpu.sync_copy(dst_hbm.at[wid], dst_v)
        pltpu.sync_copy(zeros_hbm,
                        acc.at[pl.ds(s * ROWS_PER_TILE, ROWS_PER_TILE)])
        if with_count:
            @pl.when(s == 0)
            def _():
                pltpu.sync_copy(zeros_hbm.at[pl.ds(0, CROWS)], hist_sh)
            pltpu.sync_copy(zeros_hbm.at[pl.ds(0, CROWS)], hist_v)
            for g in range(CROWS // L):
                iden_v[pl.ds(g * L, L)] = lax.iota(jnp.int32, L) + g * L
        plsc.subcore_barrier()

        def step(j, carry):
            pltpu.async_copy(y_hbm.at[src_v.at[j]], rows_v, gsem).wait()
            pltpu.sync_copy(rows_v, acc.at[dst_v.at[j]], add=True)
            if with_count:
                for g in range(CHUNK // L):
                    d16 = dst_v[j, pl.ds(g * L, L)]
                    counts, last = plsc.scan_count(d16)
                    plsc.addupdate_scatter(
                        hist_v,
                        [lax.shift_right_logical(d16, 7),
                         lax.bitwise_and(d16, 127)],
                        counts.astype(jnp.float32), mask=last)
            return carry

        lax.fori_loop(0, STEPS, step, 0)

        if with_count:
            pltpu.sync_copy(hist_v, hist_sh.at[iden_v], add=True)
        plsc.subcore_barrier()

        pltpu.sync_copy(
            acc.at[pl.ds(s * ROWS_PER_TILE, ROWS_PER_TILE)],
            out_hbm.at[c].at[pl.ds(s * ROWS_PER_TILE, ROWS_PER_TILE)])
        if with_count:
            @pl.when(s == 0)
            def _():
                pltpu.sync_copy(hist_sh, cnt_hbm.at[c])

    return sc_kernel


_sc_scatter_l1 = _make_sc_scatter(True)
_sc_scatter_l2 = _make_sc_scatter(False)


@jax.jit
def kernel(x_dict, edge_index, W_l1, b_l1, W_r1, W_l2, b_l2, W_r2):
    x = x_dict
    n_fill = E_PAD - N_EDGES
    fill = jnp.arange(n_fill, dtype=jnp.int32)
    src = jnp.concatenate([edge_index[0], fill % N_NODES])
    dst = jnp.concatenate([edge_index[1], N_NODES + fill % (N_PAD - N_NODES)])
    src = src.reshape(NW, STEPS, CHUNK)
    dst = dst.reshape(NW, STEPS, CHUNK)

    y1, z1 = _tc1(x, W_l1, W_r1, b_l1.reshape(1, D))

    zeros = jnp.zeros((ROWS_PER_TILE, D), jnp.float32)
    acc1, cnt2d = _sc_scatter_l1(y1, src, dst, zeros)
    cnt = cnt2d.reshape(NC, N_PAD)[:, :N_NODES, None]

    y2, z2, rinv = _tc2(acc1, cnt, z1, W_l2, b_l2.reshape(1, D), W_r2)

    acc2, = _sc_scatter_l2(y2, src, dst, zeros)

    return _tc3(acc2, z2, rinv)


# speedup vs baseline: 12.8809x; 1.4988x over previous
import functools

import jax
import jax.numpy as jnp
from jax import lax
from jax.experimental import pallas as pl
from jax.experimental.pallas import tpu as pltpu
from jax.experimental.pallas import tpu_sc as plsc

N_NODES = 10000
N_EDGES = 320000
D = 128
L = 16

NC = 2
NS = 16
NW = NC * NS

CHUNK = 128
RING = 4
E_PAD = 327680
STEPS = E_PAD // (NW * CHUNK)
N_PAD = 10240
ROWS_PER_TILE = N_PAD // NS
CROWS = N_PAD // CHUNK
CROWS_PER_TILE = CROWS // NS

ROW_BLOCK = 1000


def _dotT(a, w):
    return lax.dot_general(a, w, (((1,), (1,)), ((), ())),
                           preferred_element_type=jnp.float32)


def _tc1_body(x_ref, wl_ref, wr_ref, br_ref, y_ref, z_ref):
    x = x_ref[...]
    y_ref[...] = _dotT(x, wl_ref[...])
    z_ref[...] = _dotT(x, wr_ref[...]) + br_ref[...]


def _tc2_body(acc_ref, cnt_ref, z1_ref, wl2_ref, bl2_ref, wr2_ref,
              y2_ref, z2_ref, rinv_ref):
    a = acc_ref[0] + acc_ref[1]
    cnt = (cnt_ref[0] + cnt_ref[1])
    rinv = 1.0 / jnp.maximum(cnt, 1.0)
    h = jnp.maximum(a * rinv + z1_ref[...], 0.0)
    y2_ref[...] = _dotT(h, wl2_ref[...])
    z2_ref[...] = _dotT(h, wr2_ref[...]) + bl2_ref[...]
    rinv_ref[...] = jnp.broadcast_to(rinv, (ROW_BLOCK, D))


def _tc3_body(acc_ref, z2_ref, rinv_ref, out_ref):
    out_ref[...] = (acc_ref[0] + acc_ref[1]) * rinv_ref[...] + z2_ref[...]


def _tc1(x, w_l, w_r, b_r):
    grid = (N_NODES // ROW_BLOCK,)
    return pl.pallas_call(
        _tc1_body,
        grid=grid,
        in_specs=[
            pl.BlockSpec((ROW_BLOCK, D), lambda i: (i, 0)),
            pl.BlockSpec((D, D), lambda i: (0, 0)),
            pl.BlockSpec((D, D), lambda i: (0, 0)),
            pl.BlockSpec((1, D), lambda i: (0, 0)),
        ],
        out_specs=[
            pl.BlockSpec((ROW_BLOCK, D), lambda i: (i, 0)),
            pl.BlockSpec((ROW_BLOCK, D), lambda i: (i, 0)),
        ],
        out_shape=[
            jax.ShapeDtypeStruct((N_NODES, D), jnp.float32),
            jax.ShapeDtypeStruct((N_NODES, D), jnp.float32),
        ],
    )(x, w_l, w_r, b_r)


def _tc2(acc1, cnt, z1, w_l2, b_l2, w_r2):
    grid = (N_NODES // ROW_BLOCK,)
    return pl.pallas_call(
        _tc2_body,
        grid=grid,
        in_specs=[
            pl.BlockSpec((NC, ROW_BLOCK, D), lambda i: (0, i, 0)),
            pl.BlockSpec((NC, ROW_BLOCK, 1), lambda i: (0, i, 0)),
            pl.BlockSpec((ROW_BLOCK, D), lambda i: (i, 0)),
            pl.BlockSpec((D, D), lambda i: (0, 0)),
            pl.BlockSpec((1, D), lambda i: (0, 0)),
            pl.BlockSpec((D, D), lambda i: (0, 0)),
        ],
        out_specs=[
            pl.BlockSpec((ROW_BLOCK, D), lambda i: (i, 0)),
            pl.BlockSpec((ROW_BLOCK, D), lambda i: (i, 0)),
            pl.BlockSpec((ROW_BLOCK, D), lambda i: (i, 0)),
        ],
        out_shape=[
            jax.ShapeDtypeStruct((N_NODES, D), jnp.float32),
            jax.ShapeDtypeStruct((N_NODES, D), jnp.float32),
            jax.ShapeDtypeStruct((N_NODES, D), jnp.float32),
        ],
    )(acc1, cnt, z1, w_l2, b_l2, w_r2)


def _tc3(acc2, z2, rinv):
    grid = (N_NODES // ROW_BLOCK,)
    return pl.pallas_call(
        _tc3_body,
        grid=grid,
        in_specs=[
            pl.BlockSpec((NC, ROW_BLOCK, D), lambda i: (0, i, 0)),
            pl.BlockSpec((ROW_BLOCK, D), lambda i: (i, 0)),
            pl.BlockSpec((ROW_BLOCK, D), lambda i: (i, 0)),
        ],
        out_specs=pl.BlockSpec((ROW_BLOCK, D), lambda i: (i, 0)),
        out_shape=jax.ShapeDtypeStruct((N_NODES, D), jnp.float32),
    )(acc2, z2, rinv)


def _make_sc_scatter(with_count):
    mesh = plsc.VectorSubcoreMesh(core_axis_name="c", subcore_axis_name="s",
                                  num_cores=NC, num_subcores=NS)

    out_type = [jax.ShapeDtypeStruct((NC, N_PAD, D), jnp.float32)]
    scratch = [
        pltpu.VMEM((RING, CHUNK), jnp.int32),
        pltpu.VMEM((RING, CHUNK), jnp.int32),
        pltpu.VMEM((CHUNK, D), jnp.float32),
        pltpu.VMEM((CHUNK, D), jnp.float32),
        pltpu.VMEM_SHARED((N_PAD, D), jnp.float32),
        pltpu.SemaphoreType.DMA,
        pltpu.SemaphoreType.DMA,
        pltpu.SemaphoreType.DMA,
        pltpu.SemaphoreType.DMA,
    ]
    if with_count:
        out_type.append(jax.ShapeDtypeStruct((NC, CROWS, CHUNK), jnp.float32))
        scratch += [
            pltpu.VMEM((CROWS, CHUNK), jnp.float32),
            pltpu.VMEM((CROWS,), jnp.int32),
            pltpu.VMEM_SHARED((CROWS, CHUNK), jnp.float32),
        ]

    @functools.partial(
        pl.kernel, out_type=out_type, mesh=mesh, scratch_types=scratch,
        compiler_params=pltpu.CompilerParams(needs_layout_passes=False))
    def sc_kernel(y_hbm, src_hbm, dst_hbm, zeros_hbm, *refs):
        if with_count:
            (out_hbm, cnt_hbm, src_r, dst_r, rows_v0, rows_v1, acc,
             gsem0, gsem1, ssem, dsem, hist_v, iden_v, hist_sh) = refs
        else:
            (out_hbm, src_r, dst_r, rows_v0, rows_v1, acc,
             gsem0, gsem1, ssem, dsem) = refs

        c = lax.axis_index("c")
        s = lax.axis_index("s")
        wid = s * NC + c
        my_src = src_hbm.at[wid]
        my_dst = dst_hbm.at[wid]

        pltpu.sync_copy(zeros_hbm,
                        acc.at[pl.ds(s * ROWS_PER_TILE, ROWS_PER_TILE)])
        if with_count:
            @pl.when(s == 0)
            def _():
                pltpu.sync_copy(zeros_hbm.at[pl.ds(0, CROWS)], hist_sh)
            pltpu.sync_copy(zeros_hbm.at[pl.ds(0, CROWS)], hist_v)
            for g in range(CROWS // L):
                iden_v[pl.ds(g * L, L)] = lax.iota(jnp.int32, L) + g * L
        plsc.subcore_barrier()

        def start_idx(j, slot):
            pltpu.async_copy(my_src.at[j], src_r.at[slot], ssem)
            pltpu.async_copy(my_dst.at[j], dst_r.at[slot], dsem)

        def wait_idx():
            pltpu.make_async_copy(my_src.at[0], src_r.at[0], ssem).wait()
            pltpu.make_async_copy(my_dst.at[0], dst_r.at[0], dsem).wait()

        def start_gather(slot, buf, sem):
            pltpu.async_copy(y_hbm.at[src_r.at[slot]], buf, sem)

        def wait_gather(buf, sem):
            pltpu.make_async_copy(y_hbm.at[src_r.at[0]], buf, sem).wait()

        def hist_update(slot):
            for g in range(CHUNK // L):
                d16 = dst_r[slot, pl.ds(g * L, L)]
                counts, last = plsc.scan_count(d16)
                plsc.addupdate_scatter(
                    hist_v,
                    [lax.shift_right_logical(d16, 7),
                     lax.bitwise_and(d16, 127)],
                    counts.astype(jnp.float32), mask=last)

        def half_step(j, buf, sem):
            slot = lax.rem(j, RING)
            wait_idx()
            wait_gather(buf, sem)
            pltpu.sync_copy(buf, acc.at[dst_r.at[slot]], add=True)
            start_gather(lax.rem(j + 2, RING), buf, sem)
            if with_count:
                hist_update(slot)
            start_idx(jnp.minimum(j + RING, STEPS - 1), slot)

        for t in range(RING):
            start_idx(t, t)
        wait_idx()
        wait_idx()
        start_gather(0, rows_v0, gsem0)
        start_gather(1, rows_v1, gsem1)

        def step(i, carry):
            half_step(2 * i, rows_v0, gsem0)
            half_step(2 * i + 1, rows_v1, gsem1)
            return carry

        lax.fori_loop(0, STEPS // 2, step, 0)
        wait_gather(rows_v0, gsem0)
        wait_gather(rows_v1, gsem1)
        wait_idx()
        wait_idx()

        if with_count:
            pltpu.sync_copy(hist_v, hist_sh.at[iden_v], add=True)
        plsc.subcore_barrier()

        pltpu.sync_copy(
            acc.at[pl.ds(s * ROWS_PER_TILE, ROWS_PER_TILE)],
            out_hbm.at[c].at[pl.ds(s * ROWS_PER_TILE, ROWS_PER_TILE)])
        if with_count:
            @pl.when(s == 0)
            def _():
                pltpu.sync_copy(hist_sh, cnt_hbm.at[c])

    return sc_kernel


_sc_scatter_l1 = _make_sc_scatter(True)
_sc_scatter_l2 = _make_sc_scatter(False)


@jax.jit
def kernel(x_dict, edge_index, W_l1, b_l1, W_r1, W_l2, b_l2, W_r2):
    x = x_dict
    n_fill = E_PAD - N_EDGES
    fill = jnp.arange(n_fill, dtype=jnp.int32)
    src = jnp.concatenate([edge_index[0], fill % N_NODES])
    dst = jnp.concatenate([edge_index[1], N_NODES + fill % (N_PAD - N_NODES)])
    src = src.reshape(NW, STEPS, CHUNK)
    dst = dst.reshape(NW, STEPS, CHUNK)

    y1, z1 = _tc1(x, W_l1, W_r1, b_l1.reshape(1, D))

    zeros = jnp.zeros((ROWS_PER_TILE, D), jnp.float32)
    acc1, cnt2d = _sc_scatter_l1(y1, src, dst, zeros)
    cnt = cnt2d.reshape(NC, N_PAD)[:, :N_NODES, None]

    y2, z2, rinv = _tc2(acc1, cnt, z1, W_l2, b_l2.reshape(1, D), W_r2)

    acc2, = _sc_scatter_l2(y2, src, dst, zeros)

    return _tc3(acc2, z2, rinv)
